# Initial kernel scaffold; baseline (speedup 1.0000x reference)
import functools

import jax
import jax.numpy as jnp
import numpy as np
from jax import lax
from jax.experimental import pallas as pl
from jax.experimental.pallas import tpu as pltpu
from jax.experimental.pallas import tpu_sc as plsc

N = 10000
E = 320000
F = 128
D = 64
NLAYERS = 5
EMB = D * NLAYERS
G = 200
GAMMA = 0.1
LOG2 = float(np.log(2.0))

NC = 2
NS = 16
NW = NC * NS
BLK = 128
NBLK_W = 79
EP = NW * NBLK_W * BLK
NPAD = 10240
ZROWS = NPAD // NS
OROWS = N // NS


def _agg_body(z_hbm, src_hbm, dst_hbm, zeros_hbm, out_hbm,
              src_v, dst_v, rows_v, acc_sh, sem):
    c = lax.axis_index("c")
    s = lax.axis_index("s")
    wid = c * NS + s
    pltpu.sync_copy(zeros_hbm, acc_sh.at[pl.ds(s * ZROWS, ZROWS)])
    pltpu.sync_copy(src_hbm.at[wid], src_v)
    pltpu.sync_copy(dst_hbm.at[wid], dst_v)
    plsc.subcore_barrier()

    def step(j, carry):
        pltpu.async_copy(z_hbm.at[src_v.at[j]], rows_v, sem).wait()
        pltpu.sync_copy(rows_v, acc_sh.at[dst_v.at[j]], add=True)
        return carry

    lax.fori_loop(0, NBLK_W, step, 0)
    plsc.subcore_barrier()
    pltpu.sync_copy(acc_sh.at[pl.ds(s * OROWS, OROWS)],
                    out_hbm.at[c].at[pl.ds(s * OROWS, OROWS)])


_agg = pl.kernel(
    _agg_body,
    out_type=jax.ShapeDtypeStruct((NC, N, D), jnp.float32),
    mesh=plsc.VectorSubcoreMesh(core_axis_name="c", subcore_axis_name="s"),
    scratch_types=[
        pltpu.VMEM((NBLK_W, BLK), jnp.int32),
        pltpu.VMEM((NBLK_W, BLK), jnp.int32),
        pltpu.VMEM((BLK, D), jnp.float32),
        pltpu.VMEM_SHARED((NPAD, D), jnp.float32),
        pltpu.SemaphoreType.DMA,
    ],
)


def _pre_body(x_ref, w_ref, o_ref):
    o_ref[...] = jnp.dot(x_ref[...], w_ref[...],
                         preferred_element_type=jnp.float32)


_pre = pl.pallas_call(
    _pre_body,
    out_shape=jax.ShapeDtypeStruct((N, D), jnp.float32),
)


def _bn_mlp(z, a0, a1, b1, w2, b2, gm, bt):
    u = jnp.maximum(z + a0 + a1 + b1, 0.0)
    v = jnp.dot(u, w2, preferred_element_type=jnp.float32) + b2
    xr = jnp.maximum(v, 0.0)
    mean = jnp.mean(xr, axis=0, keepdims=True)
    xc = xr - mean
    var = jnp.mean(xc * xc, axis=0, keepdims=True)
    return xc / jnp.sqrt(var + 1e-5) * gm + bt


def _layer_body(z_ref, a0_ref, a1_ref, b1_ref, w2_ref, b2_ref, gm_ref,
                bt_ref, w1n_ref, x_ref, zn_ref):
    xbn = _bn_mlp(z_ref[...], a0_ref[...], a1_ref[...], b1_ref[...],
                  w2_ref[...], b2_ref[...], gm_ref[...], bt_ref[...])
    x_ref[...] = xbn
    zn_ref[...] = jnp.dot(xbn, w1n_ref[...],
                          preferred_element_type=jnp.float32)


_layer = pl.pallas_call(
    _layer_body,
    out_shape=(jax.ShapeDtypeStruct((N, D), jnp.float32),
               jax.ShapeDtypeStruct((N, D), jnp.float32)),
)


def _layer_last_body(z_ref, a0_ref, a1_ref, b1_ref, w2_ref, b2_ref, gm_ref,
                     bt_ref, x_ref):
    x_ref[...] = _bn_mlp(z_ref[...], a0_ref[...], a1_ref[...], b1_ref[...],
                         w2_ref[...], b2_ref[...], gm_ref[...], bt_ref[...])


_layer_last = pl.pallas_call(
    _layer_last_body,
    out_shape=jax.ShapeDtypeStruct((N, D), jnp.float32),
)


def _sigmoid(t):
    return 1.0 / (1.0 + jnp.exp(-t))


def _pool_body(x0, x1, x2, x3, x4, gi_ref, pn_ref,
               gw1, gb1, gw2, gb2, gsw, gsb, pw1, pb1, pw2, pb2,
               y_ref, genc_ref, prior_ref):
    M = jnp.concatenate([x0[...], x1[...], x2[...], x3[...], x4[...]],
                        axis=1)
    seg = lax.broadcasted_iota(jnp.int32, (G, N), 0)
    pt = (gi_ref[...] == seg).astype(jnp.float32)
    y = lax.dot_general(pt, M, (((1,), (0,)), ((), ())),
                        precision=lax.Precision.HIGHEST,
                        preferred_element_type=jnp.float32)
    y_ref[...] = y
    h = jnp.maximum(jnp.dot(y, gw1[...],
                            preferred_element_type=jnp.float32) + gb1[...],
                    0.0)
    h = jnp.maximum(jnp.dot(h, gw2[...],
                            preferred_element_type=jnp.float32) + gb2[...],
                    0.0)
    genc_ref[...] = h + jnp.dot(y, gsw[...],
                                preferred_element_type=jnp.float32) + gsb[...]

    def prior_d(t):
        hh = _sigmoid(jnp.dot(t, pw1[...],
                              preferred_element_type=jnp.float32) + pb1[...])
        return _sigmoid(jnp.dot(hh, pw2[...],
                                preferred_element_type=jnp.float32) + pb2[...])

    term_a = jnp.mean(jnp.log(prior_d(pn_ref[...])))
    term_b = jnp.mean(jnp.log(1.0 - prior_d(y)))
    prior_ref[0, 0] = -(term_a + term_b) * GAMMA


_pool = pl.pallas_call(
    _pool_body,
    out_shape=(jax.ShapeDtypeStruct((G, EMB), jnp.float32),
               jax.ShapeDtypeStruct((G, EMB), jnp.float32),
               jax.ShapeDtypeStruct((1, 1), jnp.float32)),
)

BLKN = 2000
NB = N // BLKN


def _softplus(t):
    return jnp.maximum(t, 0.0) + jnp.log1p(jnp.exp(-jnp.abs(t)))


def _loss_body(x0, x1, x2, x3, x4, gi_ref, genc_ref,
               lw1, lb1, lw2, lb2, lsw, lsb, epos_ref, eneg_ref):
    i = pl.program_id(0)
    Mb = jnp.concatenate([x0[...], x1[...], x2[...], x3[...], x4[...]],
                         axis=1)
    h = jnp.maximum(jnp.dot(Mb, lw1[...],
                            preferred_element_type=jnp.float32) + lb1[...],
                    0.0)
    h = jnp.maximum(jnp.dot(h, lw2[...],
                            preferred_element_type=jnp.float32) + lb2[...],
                    0.0)
    lenc = h + jnp.dot(Mb, lsw[...],
                       preferred_element_type=jnp.float32) + lsb[...]
    res = lax.dot_general(lenc, genc_ref[...], (((1,), (1,)), ((), ())),
                          preferred_element_type=jnp.float32)
    seg = lax.broadcasted_iota(jnp.int32, (BLKN, G), 1)
    pos = (gi_ref[...] == seg).astype(jnp.float32)
    rp = res * pos
    ep = jnp.sum(LOG2 - _softplus(-rp))
    qn = res * (1.0 - pos)
    en = jnp.sum(_softplus(-qn) + qn - LOG2)

    @pl.when(i == 0)
    def _init():
        epos_ref[0, 0] = 0.0
        eneg_ref[0, 0] = 0.0

    epos_ref[0, 0] += ep
    eneg_ref[0, 0] += en


_loss = pl.pallas_call(
    _loss_body,
    grid=(NB,),
    in_specs=[pl.BlockSpec((BLKN, D), lambda i: (i, 0))] * 5
    + [pl.BlockSpec((BLKN, 1), lambda i: (i, 0)),
       pl.BlockSpec((G, EMB), lambda i: (0, 0)),
       pl.BlockSpec((EMB, EMB), lambda i: (0, 0)),
       pl.BlockSpec((1, EMB), lambda i: (0, 0)),
       pl.BlockSpec((EMB, EMB), lambda i: (0, 0)),
       pl.BlockSpec((1, EMB), lambda i: (0, 0)),
       pl.BlockSpec((EMB, EMB), lambda i: (0, 0)),
       pl.BlockSpec((1, EMB), lambda i: (0, 0))],
    out_specs=(pl.BlockSpec((1, 1), lambda i: (0, 0)),
               pl.BlockSpec((1, 1), lambda i: (0, 0))),
    out_shape=(jax.ShapeDtypeStruct((1, 1), jnp.float32),
               jax.ShapeDtypeStruct((1, 1), jnp.float32)),
)


def kernel(node_features, edge_index, graph_index, prior_noise, params):
    p = params
    src = edge_index[0].astype(jnp.int32)
    dst = edge_index[1].astype(jnp.int32)
    pad = EP - E
    pad_i = jnp.arange(pad, dtype=jnp.int32)
    src2 = jnp.concatenate([src, pad_i % N]).reshape(NW, NBLK_W, BLK)
    dst2 = jnp.concatenate([dst, N + (pad_i % 16)]).reshape(NW, NBLK_W, BLK)
    zeros_hbm = jnp.zeros((ZROWS, D), jnp.float32)
    gi = graph_index.astype(jnp.int32)
    gi_row = gi.reshape(1, N)
    gi_col = gi.reshape(N, 1)

    z = _pre(node_features, p['conv0_w1'])
    xs = []
    for l in range(NLAYERS):
        apart = _agg(z, src2, dst2, zeros_hbm)
        b1 = p['conv%d_b1' % l].reshape(1, D)
        w2 = p['conv%d_w2' % l]
        b2 = p['conv%d_b2' % l].reshape(1, D)
        gm = p['bn%d_gamma' % l].reshape(1, D)
        bt = p['bn%d_beta' % l].reshape(1, D)
        if l < NLAYERS - 1:
            x, z = _layer(z, apart[0], apart[1], b1, w2, b2, gm, bt,
                          p['conv%d_w1' % (l + 1)])
        else:
            x = _layer_last(z, apart[0], apart[1], b1, w2, b2, gm, bt)
        xs.append(x)

    y, genc, prior = _pool(
        xs[0], xs[1], xs[2], xs[3], xs[4], gi_row, prior_noise,
        p['gd_w1'], p['gd_b1'].reshape(1, EMB),
        p['gd_w2'], p['gd_b2'].reshape(1, EMB),
        p['gd_skip_w'], p['gd_skip_b'].reshape(1, EMB),
        p['pd_w1'], p['pd_b1'].reshape(1, EMB),
        p['pd_w2'], p['pd_b2'].reshape(1, 1))
    epos, eneg = _loss(
        xs[0], xs[1], xs[2], xs[3], xs[4], gi_col, genc,
        p['ld_w1'], p['ld_b1'].reshape(1, EMB),
        p['ld_w2'], p['ld_b2'].reshape(1, EMB),
        p['ld_skip_w'], p['ld_skip_b'].reshape(1, EMB))
    e_pos = epos[0, 0] / N
    e_neg = eneg[0, 0] / (N * (G - 1))
    return (e_neg - e_pos) + prior[0, 0]


# speedup vs baseline: 7.6196x; 7.6196x over previous
import functools

import jax
import jax.numpy as jnp
import numpy as np
from jax import lax
from jax.experimental import pallas as pl
from jax.experimental.pallas import tpu as pltpu
from jax.experimental.pallas import tpu_sc as plsc

N = 10000
E = 320000
F = 128
D = 64
NLAYERS = 5
EMB = D * NLAYERS
G = 200
GAMMA = 0.1
LOG2 = float(np.log(2.0))

NC = 2
NS = 16
NW = NC * NS
BLK = 128
NBLK_W = 79
EP = NW * NBLK_W * BLK
NPAD = 10240
ZROWS = NPAD // NS
OROWS = N // NS


def _agg_body(z_hbm, src_hbm, dst_hbm, zeros_hbm, out_hbm,
              src_v, dst_v, rows_v, acc_sh, sem):
    c = lax.axis_index("c")
    s = lax.axis_index("s")
    wid = c * NS + s
    pltpu.sync_copy(zeros_hbm, acc_sh.at[pl.ds(s * ZROWS, ZROWS)])
    pltpu.sync_copy(src_hbm.at[wid], src_v)
    pltpu.sync_copy(dst_hbm.at[wid], dst_v)
    plsc.subcore_barrier()

    def step(j, carry):
        pltpu.async_copy(z_hbm.at[src_v.at[j]], rows_v, sem).wait()
        pltpu.sync_copy(rows_v, acc_sh.at[dst_v.at[j]], add=True)
        return carry

    lax.fori_loop(0, NBLK_W, step, 0)
    plsc.subcore_barrier()
    pltpu.sync_copy(acc_sh.at[pl.ds(s * ZROWS, ZROWS)],
                    out_hbm.at[c].at[pl.ds(s * ZROWS, ZROWS)])


_agg = pl.kernel(
    _agg_body,
    out_type=jax.ShapeDtypeStruct((NC, NPAD, D), jnp.float32),
    mesh=plsc.VectorSubcoreMesh(core_axis_name="c", subcore_axis_name="s"),
    scratch_types=[
        pltpu.VMEM((NBLK_W, BLK), jnp.int32),
        pltpu.VMEM((NBLK_W, BLK), jnp.int32),
        pltpu.VMEM((BLK, D), jnp.float32),
        pltpu.VMEM_SHARED((NPAD, D), jnp.float32),
        pltpu.SemaphoreType.DMA,
    ],
    compiler_params=pltpu.CompilerParams(use_tc_tiling_on_sc=False),
)


def _pre_body(x_ref, w_ref, o_ref):
    o_ref[...] = jnp.dot(x_ref[...], w_ref[...],
                         preferred_element_type=jnp.float32)


_pre = pl.pallas_call(
    _pre_body,
    out_shape=jax.ShapeDtypeStruct((N, D), jnp.float32),
)


def _bn_mlp(z, a0, a1, b1, w2, b2, gm, bt):
    u = jnp.maximum(z + a0 + a1 + b1, 0.0)
    v = jnp.dot(u, w2, preferred_element_type=jnp.float32) + b2
    xr = jnp.maximum(v, 0.0)
    mean = jnp.mean(xr, axis=0, keepdims=True)
    xc = xr - mean
    var = jnp.mean(xc * xc, axis=0, keepdims=True)
    return xc / jnp.sqrt(var + 1e-5) * gm + bt


def _layer_body(z_ref, a0_ref, a1_ref, b1_ref, w2_ref, b2_ref, gm_ref,
                bt_ref, w1n_ref, x_ref, zn_ref):
    xbn = _bn_mlp(z_ref[...], a0_ref[...], a1_ref[...], b1_ref[...],
                  w2_ref[...], b2_ref[...], gm_ref[...], bt_ref[...])
    x_ref[...] = xbn
    zn_ref[...] = jnp.dot(xbn, w1n_ref[...],
                          preferred_element_type=jnp.float32)


_layer = pl.pallas_call(
    _layer_body,
    out_shape=(jax.ShapeDtypeStruct((N, D), jnp.float32),
               jax.ShapeDtypeStruct((N, D), jnp.float32)),
)


def _layer_last_body(z_ref, a0_ref, a1_ref, b1_ref, w2_ref, b2_ref, gm_ref,
                     bt_ref, x_ref):
    x_ref[...] = _bn_mlp(z_ref[...], a0_ref[...], a1_ref[...], b1_ref[...],
                         w2_ref[...], b2_ref[...], gm_ref[...], bt_ref[...])


_layer_last = pl.pallas_call(
    _layer_last_body,
    out_shape=jax.ShapeDtypeStruct((N, D), jnp.float32),
)


def _sigmoid(t):
    return 1.0 / (1.0 + jnp.exp(-t))


def _pool_body(x0, x1, x2, x3, x4, gi_ref, pn_ref,
               gw1, gb1, gw2, gb2, gsw, gsb, pw1, pb1, pw2, pb2,
               y_ref, genc_ref, prior_ref):
    M = jnp.concatenate([x0[...], x1[...], x2[...], x3[...], x4[...]],
                        axis=1)
    seg = lax.broadcasted_iota(jnp.int32, (G, N), 0)
    pt = (gi_ref[...] == seg).astype(jnp.float32)
    y = lax.dot_general(pt, M, (((1,), (0,)), ((), ())),
                        precision=lax.Precision.HIGHEST,
                        preferred_element_type=jnp.float32)
    y_ref[...] = y
    h = jnp.maximum(jnp.dot(y, gw1[...],
                            preferred_element_type=jnp.float32) + gb1[...],
                    0.0)
    h = jnp.maximum(jnp.dot(h, gw2[...],
                            preferred_element_type=jnp.float32) + gb2[...],
                    0.0)
    genc_ref[...] = h + jnp.dot(y, gsw[...],
                                preferred_element_type=jnp.float32) + gsb[...]

    def prior_d(t):
        hh = _sigmoid(jnp.dot(t, pw1[...],
                              preferred_element_type=jnp.float32) + pb1[...])
        return _sigmoid(jnp.dot(hh, pw2[...],
                                preferred_element_type=jnp.float32) + pb2[...])

    term_a = jnp.mean(jnp.log(prior_d(pn_ref[...])))
    term_b = jnp.mean(jnp.log(1.0 - prior_d(y)))
    prior_ref[...] = jnp.full((1, 1), -(term_a + term_b) * GAMMA,
                              jnp.float32)


_pool = pl.pallas_call(
    _pool_body,
    out_shape=(jax.ShapeDtypeStruct((G, EMB), jnp.float32),
               jax.ShapeDtypeStruct((G, EMB), jnp.float32),
               jax.ShapeDtypeStruct((1, 1), jnp.float32)),
)

BLKN = 2000
NB = N // BLKN


def _softplus(t):
    return jnp.maximum(t, 0.0) + jnp.log1p(jnp.exp(-jnp.abs(t)))


def _loss_body(x0, x1, x2, x3, x4, gi_ref, genc_ref,
               lw1, lb1, lw2, lb2, lsw, lsb, epos_ref, eneg_ref):
    i = pl.program_id(0)
    Mb = jnp.concatenate([x0[...], x1[...], x2[...], x3[...], x4[...]],
                         axis=1)
    h = jnp.maximum(jnp.dot(Mb, lw1[...],
                            preferred_element_type=jnp.float32) + lb1[...],
                    0.0)
    h = jnp.maximum(jnp.dot(h, lw2[...],
                            preferred_element_type=jnp.float32) + lb2[...],
                    0.0)
    lenc = h + jnp.dot(Mb, lsw[...],
                       preferred_element_type=jnp.float32) + lsb[...]
    res = lax.dot_general(lenc, genc_ref[...], (((1,), (1,)), ((), ())),
                          preferred_element_type=jnp.float32)
    seg = lax.broadcasted_iota(jnp.int32, (BLKN, G), 1)
    pos = (gi_ref[...] == seg).astype(jnp.float32)
    rp = res * pos
    ep = jnp.sum(LOG2 - _softplus(-rp))
    qn = res * (1.0 - pos)
    en = jnp.sum(_softplus(-qn) + qn - LOG2)

    @pl.when(i == 0)
    def _init():
        epos_ref[...] = jnp.zeros((1, 1), jnp.float32)
        eneg_ref[...] = jnp.zeros((1, 1), jnp.float32)

    epos_ref[...] = epos_ref[...] + ep
    eneg_ref[...] = eneg_ref[...] + en


_loss = pl.pallas_call(
    _loss_body,
    grid=(NB,),
    in_specs=[pl.BlockSpec((BLKN, D), lambda i: (i, 0))] * 5
    + [pl.BlockSpec((BLKN, 1), lambda i: (i, 0)),
       pl.BlockSpec((G, EMB), lambda i: (0, 0)),
       pl.BlockSpec((EMB, EMB), lambda i: (0, 0)),
       pl.BlockSpec((1, EMB), lambda i: (0, 0)),
       pl.BlockSpec((EMB, EMB), lambda i: (0, 0)),
       pl.BlockSpec((1, EMB), lambda i: (0, 0)),
       pl.BlockSpec((EMB, EMB), lambda i: (0, 0)),
       pl.BlockSpec((1, EMB), lambda i: (0, 0))],
    out_specs=(pl.BlockSpec((1, 1), lambda i: (0, 0)),
               pl.BlockSpec((1, 1), lambda i: (0, 0))),
    out_shape=(jax.ShapeDtypeStruct((1, 1), jnp.float32),
               jax.ShapeDtypeStruct((1, 1), jnp.float32)),
)


def kernel(node_features, edge_index, graph_index, prior_noise, params):
    p = params
    src = edge_index[0].astype(jnp.int32)
    dst = edge_index[1].astype(jnp.int32)
    pad = EP - E
    pad_i = jnp.arange(pad, dtype=jnp.int32)
    src2 = jnp.concatenate([src, pad_i % N]).reshape(NW, NBLK_W, BLK)
    dst2 = jnp.concatenate([dst, N + (pad_i % 16)]).reshape(NW, NBLK_W, BLK)
    zeros_hbm = jnp.zeros((ZROWS, D), jnp.float32)
    gi = graph_index.astype(jnp.int32)
    gi_row = gi.reshape(1, N)
    gi_col = gi.reshape(N, 1)

    z = _pre(node_features, p['conv0_w1'])
    xs = []
    for l in range(NLAYERS):
        apart = _agg(z, src2, dst2, zeros_hbm)[:, :N, :]
        b1 = p['conv%d_b1' % l].reshape(1, D)
        w2 = p['conv%d_w2' % l]
        b2 = p['conv%d_b2' % l].reshape(1, D)
        gm = p['bn%d_gamma' % l].reshape(1, D)
        bt = p['bn%d_beta' % l].reshape(1, D)
        if l < NLAYERS - 1:
            x, z = _layer(z, apart[0], apart[1], b1, w2, b2, gm, bt,
                          p['conv%d_w1' % (l + 1)])
        else:
            x = _layer_last(z, apart[0], apart[1], b1, w2, b2, gm, bt)
        xs.append(x)

    y, genc, prior = _pool(
        xs[0], xs[1], xs[2], xs[3], xs[4], gi_row, prior_noise,
        p['gd_w1'], p['gd_b1'].reshape(1, EMB),
        p['gd_w2'], p['gd_b2'].reshape(1, EMB),
        p['gd_skip_w'], p['gd_skip_b'].reshape(1, EMB),
        p['pd_w1'], p['pd_b1'].reshape(1, EMB),
        p['pd_w2'], p['pd_b2'].reshape(1, 1))
    epos, eneg = _loss(
        xs[0], xs[1], xs[2], xs[3], xs[4], gi_col, genc,
        p['ld_w1'], p['ld_b1'].reshape(1, EMB),
        p['ld_w2'], p['ld_b2'].reshape(1, EMB),
        p['ld_skip_w'], p['ld_skip_b'].reshape(1, EMB))
    e_pos = epos[0, 0] / N
    e_neg = eneg[0, 0] / (N * (G - 1))
    return (e_neg - e_pos) + prior[0, 0]


# speedup vs baseline: 12.4840x; 1.6384x over previous
import functools

import jax
import jax.numpy as jnp
import numpy as np
from jax import lax
from jax.experimental import pallas as pl
from jax.experimental.pallas import tpu as pltpu
from jax.experimental.pallas import tpu_sc as plsc

N = 10000
E = 320000
F = 128
D = 64
NLAYERS = 5
EMB = D * NLAYERS
G = 200
GAMMA = 0.1
LOG2 = float(np.log(2.0))

NC = 2
NS = 16
NW = NC * NS
BLK = 128
NBLK_W = 80
EP = NW * NBLK_W * BLK
NPAD = 10240
ZROWS = NPAD // NS
SROWS = N // NS
NBUF = 8
NGRP = NBLK_W // NBUF


def _agg_body(z_hbm, src_hbm, dst_hbm, out_hbm,
              src_v, dst_v, rows_v, acc_sh, sem):
    c = lax.axis_index("c")
    s = lax.axis_index("s")
    wid = c * NS + s
    zv = jnp.zeros((16,), jnp.float32)

    def zrow(r, carry):
        for k in range(D // 16):
            rows_v[0, r, pl.ds(k * 16, 16)] = zv
        return carry

    lax.fori_loop(0, BLK, zrow, 0)
    for t in range(ZROWS // BLK):
        pltpu.sync_copy(rows_v.at[0],
                        acc_sh.at[pl.ds(s * ZROWS + t * BLK, BLK)])
    pltpu.sync_copy(src_hbm.at[wid], src_v)
    pltpu.sync_copy(dst_hbm.at[wid], dst_v)
    plsc.subcore_barrier()

    for b in range(NBUF):
        pltpu.async_copy(z_hbm.at[src_v.at[b]], rows_v.at[b], sem)

    def group(t, carry):
        base = t * NBUF
        for b in range(NBUF):
            j = base + b
            pltpu.make_async_copy(z_hbm.at[src_v.at[j]], rows_v.at[b],
                                  sem).wait()
            pltpu.sync_copy(rows_v.at[b], acc_sh.at[dst_v.at[j]], add=True)

            @pl.when(j + NBUF < NBLK_W)
            def _refill():
                pltpu.async_copy(z_hbm.at[src_v.at[j + NBUF]], rows_v.at[b],
                                 sem)

        return carry

    lax.fori_loop(0, NGRP, group, 0)
    plsc.subcore_barrier()
    pltpu.sync_copy(acc_sh.at[pl.ds(s * ZROWS, ZROWS)],
                    out_hbm.at[c].at[pl.ds(s * ZROWS, ZROWS)])


_agg = pl.kernel(
    _agg_body,
    out_type=jax.ShapeDtypeStruct((NC, NPAD, D), jnp.float32),
    mesh=plsc.VectorSubcoreMesh(core_axis_name="c", subcore_axis_name="s"),
    scratch_types=[
        pltpu.VMEM((NBLK_W, BLK), jnp.int32),
        pltpu.VMEM((NBLK_W, BLK), jnp.int32),
        pltpu.VMEM((NBUF, BLK, D), jnp.float32),
        pltpu.VMEM_SHARED((NPAD, D), jnp.float32),
        pltpu.SemaphoreType.DMA,
    ],
    compiler_params=pltpu.CompilerParams(use_tc_tiling_on_sc=False),
)


def _pre_body(x_ref, w_ref, o_ref):
    o_ref[...] = jnp.dot(x_ref[...], w_ref[...],
                         preferred_element_type=jnp.float32)


_pre = pl.pallas_call(
    _pre_body,
    out_shape=jax.ShapeDtypeStruct((N, D), jnp.float32),
)


def _bn_mlp(z, a0, a1, b1, w2, b2, gm, bt):
    u = jnp.maximum(z + a0 + a1 + b1, 0.0)
    v = jnp.dot(u, w2, preferred_element_type=jnp.float32) + b2
    xr = jnp.maximum(v, 0.0)
    mean = jnp.mean(xr, axis=0, keepdims=True)
    xc = xr - mean
    var = jnp.mean(xc * xc, axis=0, keepdims=True)
    return xc / jnp.sqrt(var + 1e-5) * gm + bt


def _layer_body(z_ref, a0_ref, a1_ref, b1_ref, w2_ref, b2_ref, gm_ref,
                bt_ref, w1n_ref, x_ref, zn_ref):
    xbn = _bn_mlp(z_ref[...], a0_ref[...], a1_ref[...], b1_ref[...],
                  w2_ref[...], b2_ref[...], gm_ref[...], bt_ref[...])
    x_ref[...] = xbn
    zn_ref[...] = jnp.dot(xbn, w1n_ref[...],
                          preferred_element_type=jnp.float32)


_layer = pl.pallas_call(
    _layer_body,
    out_shape=(jax.ShapeDtypeStruct((N, D), jnp.float32),
               jax.ShapeDtypeStruct((N, D), jnp.float32)),
)


def _layer_last_body(z_ref, a0_ref, a1_ref, b1_ref, w2_ref, b2_ref, gm_ref,
                     bt_ref, x_ref):
    x_ref[...] = _bn_mlp(z_ref[...], a0_ref[...], a1_ref[...], b1_ref[...],
                         w2_ref[...], b2_ref[...], gm_ref[...], bt_ref[...])


_layer_last = pl.pallas_call(
    _layer_last_body,
    out_shape=jax.ShapeDtypeStruct((N, D), jnp.float32),
)


def _sigmoid(t):
    return 1.0 / (1.0 + jnp.exp(-t))


def _pool_body(x0, x1, x2, x3, x4, gi_ref, pn_ref,
               gw1, gb1, gw2, gb2, gsw, gsb, pw1, pb1, pw2, pb2,
               y_ref, genc_ref, prior_ref):
    M = jnp.concatenate([x0[...], x1[...], x2[...], x3[...], x4[...]],
                        axis=1)
    seg = lax.broadcasted_iota(jnp.int32, (G, N), 0)
    pt = (gi_ref[...] == seg).astype(jnp.float32)
    y = lax.dot_general(pt, M, (((1,), (0,)), ((), ())),
                        precision=lax.Precision.HIGHEST,
                        preferred_element_type=jnp.float32)
    y_ref[...] = y
    h = jnp.maximum(jnp.dot(y, gw1[...],
                            preferred_element_type=jnp.float32) + gb1[...],
                    0.0)
    h = jnp.maximum(jnp.dot(h, gw2[...],
                            preferred_element_type=jnp.float32) + gb2[...],
                    0.0)
    genc_ref[...] = h + jnp.dot(y, gsw[...],
                                preferred_element_type=jnp.float32) + gsb[...]

    def prior_d(t):
        hh = _sigmoid(jnp.dot(t, pw1[...],
                              preferred_element_type=jnp.float32) + pb1[...])
        return _sigmoid(jnp.dot(hh, pw2[...],
                                preferred_element_type=jnp.float32) + pb2[...])

    term_a = jnp.mean(jnp.log(prior_d(pn_ref[...])))
    term_b = jnp.mean(jnp.log(1.0 - prior_d(y)))
    prior_ref[...] = jnp.full((1, 1), -(term_a + term_b) * GAMMA,
                              jnp.float32)


_pool = pl.pallas_call(
    _pool_body,
    out_shape=(jax.ShapeDtypeStruct((G, EMB), jnp.float32),
               jax.ShapeDtypeStruct((G, EMB), jnp.float32),
               jax.ShapeDtypeStruct((1, 1), jnp.float32)),
)

BLKN = 2000
NB = N // BLKN


def _softplus(t):
    return jnp.maximum(t, 0.0) + jnp.log1p(jnp.exp(-jnp.abs(t)))


def _loss_body(x0, x1, x2, x3, x4, gi_ref, genc_ref,
               lw1, lb1, lw2, lb2, lsw, lsb, epos_ref, eneg_ref):
    i = pl.program_id(0)
    Mb = jnp.concatenate([x0[...], x1[...], x2[...], x3[...], x4[...]],
                         axis=1)
    h = jnp.maximum(jnp.dot(Mb, lw1[...],
                            preferred_element_type=jnp.float32) + lb1[...],
                    0.0)
    h = jnp.maximum(jnp.dot(h, lw2[...],
                            preferred_element_type=jnp.float32) + lb2[...],
                    0.0)
    lenc = h + jnp.dot(Mb, lsw[...],
                       preferred_element_type=jnp.float32) + lsb[...]
    res = lax.dot_general(lenc, genc_ref[...], (((1,), (1,)), ((), ())),
                          preferred_element_type=jnp.float32)
    seg = lax.broadcasted_iota(jnp.int32, (BLKN, G), 1)
    pos = (gi_ref[...] == seg).astype(jnp.float32)
    rp = res * pos
    ep = jnp.sum(LOG2 - _softplus(-rp))
    qn = res * (1.0 - pos)
    en = jnp.sum(_softplus(-qn) + qn - LOG2)

    @pl.when(i == 0)
    def _init():
        epos_ref[...] = jnp.zeros((1, 1), jnp.float32)
        eneg_ref[...] = jnp.zeros((1, 1), jnp.float32)

    epos_ref[...] = epos_ref[...] + ep
    eneg_ref[...] = eneg_ref[...] + en


_loss = pl.pallas_call(
    _loss_body,
    grid=(NB,),
    in_specs=[pl.BlockSpec((BLKN, D), lambda i: (i, 0))] * 5
    + [pl.BlockSpec((BLKN, 1), lambda i: (i, 0)),
       pl.BlockSpec((G, EMB), lambda i: (0, 0)),
       pl.BlockSpec((EMB, EMB), lambda i: (0, 0)),
       pl.BlockSpec((1, EMB), lambda i: (0, 0)),
       pl.BlockSpec((EMB, EMB), lambda i: (0, 0)),
       pl.BlockSpec((1, EMB), lambda i: (0, 0)),
       pl.BlockSpec((EMB, EMB), lambda i: (0, 0)),
       pl.BlockSpec((1, EMB), lambda i: (0, 0))],
    out_specs=(pl.BlockSpec((1, 1), lambda i: (0, 0)),
               pl.BlockSpec((1, 1), lambda i: (0, 0))),
    out_shape=(jax.ShapeDtypeStruct((1, 1), jnp.float32),
               jax.ShapeDtypeStruct((1, 1), jnp.float32)),
)


def kernel(node_features, edge_index, graph_index, prior_noise, params):
    p = params
    src = edge_index[0].astype(jnp.int32)
    dst = edge_index[1].astype(jnp.int32)
    pad = EP - E
    pad_i = jnp.arange(pad, dtype=jnp.int32)
    src2 = jnp.concatenate([src, pad_i % N]).reshape(NW, NBLK_W, BLK)
    dst2 = jnp.concatenate([dst, N + (pad_i % 16)]).reshape(NW, NBLK_W, BLK)
    gi = graph_index.astype(jnp.int32)
    gi_row = gi.reshape(1, N)
    gi_col = gi.reshape(N, 1)

    z = _pre(node_features, p['conv0_w1'])
    xs = []
    for l in range(NLAYERS):
        apart = _agg(z, src2, dst2)[:, :N, :]
        b1 = p['conv%d_b1' % l].reshape(1, D)
        w2 = p['conv%d_w2' % l]
        b2 = p['conv%d_b2' % l].reshape(1, D)
        gm = p['bn%d_gamma' % l].reshape(1, D)
        bt = p['bn%d_beta' % l].reshape(1, D)
        if l < NLAYERS - 1:
            x, z = _layer(z, apart[0], apart[1], b1, w2, b2, gm, bt,
                          p['conv%d_w1' % (l + 1)])
        else:
            x = _layer_last(z, apart[0], apart[1], b1, w2, b2, gm, bt)
        xs.append(x)

    y, genc, prior = _pool(
        xs[0], xs[1], xs[2], xs[3], xs[4], gi_row, prior_noise,
        p['gd_w1'], p['gd_b1'].reshape(1, EMB),
        p['gd_w2'], p['gd_b2'].reshape(1, EMB),
        p['gd_skip_w'], p['gd_skip_b'].reshape(1, EMB),
        p['pd_w1'], p['pd_b1'].reshape(1, EMB),
        p['pd_w2'], p['pd_b2'].reshape(1, 1))
    epos, eneg = _loss(
        xs[0], xs[1], xs[2], xs[3], xs[4], gi_col, genc,
        p['ld_w1'], p['ld_b1'].reshape(1, EMB),
        p['ld_w2'], p['ld_b2'].reshape(1, EMB),
        p['ld_skip_w'], p['ld_skip_b'].reshape(1, EMB))
    e_pos = epos[0, 0] / N
    e_neg = eneg[0, 0] / (N * (G - 1))
    return (e_neg - e_pos) + prior[0, 0]


# speedup vs baseline: 12.5329x; 1.0039x over previous
import functools

import jax
import jax.numpy as jnp
import numpy as np
from jax import lax
from jax.experimental import pallas as pl
from jax.experimental.pallas import tpu as pltpu
from jax.experimental.pallas import tpu_sc as plsc

N = 10000
E = 320000
F = 128
D = 64
NLAYERS = 5
EMB = D * NLAYERS
G = 200
GAMMA = 0.1
LOG2 = float(np.log(2.0))

NC = 2
NS = 16
NW = NC * NS
BLK = 128
NBLK_W = 80
EP = NW * NBLK_W * BLK
NPAD = 10240
ZROWS = NPAD // NS
SROWS = N // NS
NBUF = 8
NGRP = NBLK_W // NBUF


def _agg_body(z_hbm, src_hbm, dst_hbm, out_hbm,
              src_v, dst_v, rows_v, acc_sh, gsem, ssem):
    c = lax.axis_index("c")
    s = lax.axis_index("s")
    wid = c * NS + s
    zv = jnp.zeros((16,), jnp.float32)

    def zrow(r, carry):
        for k in range(D // 16):
            rows_v[0, r, pl.ds(k * 16, 16)] = zv
        return carry

    lax.fori_loop(0, BLK, zrow, 0)
    for t in range(ZROWS // BLK):
        pltpu.sync_copy(rows_v.at[0],
                        acc_sh.at[pl.ds(s * ZROWS + t * BLK, BLK)])
    pltpu.sync_copy(src_hbm.at[wid], src_v)
    pltpu.sync_copy(dst_hbm.at[wid], dst_v)
    plsc.subcore_barrier()

    for b in range(NBUF):
        pltpu.async_copy(z_hbm.at[src_v.at[b]], rows_v.at[b], gsem)

    def step(j, carry):
        b = lax.rem(j, NBUF)
        pltpu.make_async_copy(z_hbm.at[src_v.at[j]], rows_v.at[b],
                              gsem).wait()
        pltpu.async_copy(rows_v.at[b], acc_sh.at[dst_v.at[j]], ssem,
                         add=True)
        k = j - 1

        @pl.when(jnp.logical_and(k >= 0, k + NBUF < NBLK_W))
        def _refill():
            kb = lax.rem(k, NBUF)
            pltpu.make_async_copy(rows_v.at[kb], acc_sh.at[dst_v.at[k]],
                                  ssem).wait()
            pltpu.async_copy(z_hbm.at[src_v.at[k + NBUF]], rows_v.at[kb],
                             gsem)

        return carry

    lax.fori_loop(0, NBLK_W, step, 0)

    def drain(j, carry):
        b = lax.rem(j, NBUF)
        pltpu.make_async_copy(rows_v.at[b], acc_sh.at[dst_v.at[j]],
                              ssem).wait()
        return carry

    lax.fori_loop(NBLK_W - NBUF, NBLK_W, drain, 0)
    plsc.subcore_barrier()
    pltpu.sync_copy(acc_sh.at[pl.ds(s * ZROWS, ZROWS)],
                    out_hbm.at[c].at[pl.ds(s * ZROWS, ZROWS)])


_agg = pl.kernel(
    _agg_body,
    out_type=jax.ShapeDtypeStruct((NC, NPAD, D), jnp.float32),
    mesh=plsc.VectorSubcoreMesh(core_axis_name="c", subcore_axis_name="s"),
    scratch_types=[
        pltpu.VMEM((NBLK_W, BLK), jnp.int32),
        pltpu.VMEM((NBLK_W, BLK), jnp.int32),
        pltpu.VMEM((NBUF, BLK, D), jnp.float32),
        pltpu.VMEM_SHARED((NPAD, D), jnp.float32),
        pltpu.SemaphoreType.DMA,
        pltpu.SemaphoreType.DMA,
    ],
    compiler_params=pltpu.CompilerParams(use_tc_tiling_on_sc=False),
)


def _pre_body(x_ref, w_ref, o_ref):
    o_ref[...] = jnp.dot(x_ref[...], w_ref[...],
                         preferred_element_type=jnp.float32)


_pre = pl.pallas_call(
    _pre_body,
    out_shape=jax.ShapeDtypeStruct((N, D), jnp.float32),
)


def _bn_mlp(z, a0, a1, b1, w2, b2, gm, bt):
    u = jnp.maximum(z + a0 + a1 + b1, 0.0)
    v = jnp.dot(u, w2, preferred_element_type=jnp.float32) + b2
    xr = jnp.maximum(v, 0.0)
    mean = jnp.mean(xr, axis=0, keepdims=True)
    xc = xr - mean
    var = jnp.mean(xc * xc, axis=0, keepdims=True)
    return xc / jnp.sqrt(var + 1e-5) * gm + bt


def _layer_body(z_ref, a0_ref, a1_ref, b1_ref, w2_ref, b2_ref, gm_ref,
                bt_ref, w1n_ref, x_ref, zn_ref):
    xbn = _bn_mlp(z_ref[...], a0_ref[...], a1_ref[...], b1_ref[...],
                  w2_ref[...], b2_ref[...], gm_ref[...], bt_ref[...])
    x_ref[...] = xbn
    zn_ref[...] = jnp.dot(xbn, w1n_ref[...],
                          preferred_element_type=jnp.float32)


_layer = pl.pallas_call(
    _layer_body,
    out_shape=(jax.ShapeDtypeStruct((N, D), jnp.float32),
               jax.ShapeDtypeStruct((N, D), jnp.float32)),
)


def _layer_last_body(z_ref, a0_ref, a1_ref, b1_ref, w2_ref, b2_ref, gm_ref,
                     bt_ref, x_ref):
    x_ref[...] = _bn_mlp(z_ref[...], a0_ref[...], a1_ref[...], b1_ref[...],
                         w2_ref[...], b2_ref[...], gm_ref[...], bt_ref[...])


_layer_last = pl.pallas_call(
    _layer_last_body,
    out_shape=jax.ShapeDtypeStruct((N, D), jnp.float32),
)


def _sigmoid(t):
    return 1.0 / (1.0 + jnp.exp(-t))


def _pool_body(x0, x1, x2, x3, x4, gi_ref, pn_ref,
               gw1, gb1, gw2, gb2, gsw, gsb, pw1, pb1, pw2, pb2,
               y_ref, genc_ref, prior_ref):
    M = jnp.concatenate([x0[...], x1[...], x2[...], x3[...], x4[...]],
                        axis=1)
    seg = lax.broadcasted_iota(jnp.int32, (G, N), 0)
    pt = (gi_ref[...] == seg).astype(jnp.float32)
    y = lax.dot_general(pt, M, (((1,), (0,)), ((), ())),
                        precision=lax.Precision.HIGHEST,
                        preferred_element_type=jnp.float32)
    y_ref[...] = y
    h = jnp.maximum(jnp.dot(y, gw1[...],
                            preferred_element_type=jnp.float32) + gb1[...],
                    0.0)
    h = jnp.maximum(jnp.dot(h, gw2[...],
                            preferred_element_type=jnp.float32) + gb2[...],
                    0.0)
    genc_ref[...] = h + jnp.dot(y, gsw[...],
                                preferred_element_type=jnp.float32) + gsb[...]

    def prior_d(t):
        hh = _sigmoid(jnp.dot(t, pw1[...],
                              preferred_element_type=jnp.float32) + pb1[...])
        return _sigmoid(jnp.dot(hh, pw2[...],
                                preferred_element_type=jnp.float32) + pb2[...])

    term_a = jnp.mean(jnp.log(prior_d(pn_ref[...])))
    term_b = jnp.mean(jnp.log(1.0 - prior_d(y)))
    prior_ref[...] = jnp.full((1, 1), -(term_a + term_b) * GAMMA,
                              jnp.float32)


_pool = pl.pallas_call(
    _pool_body,
    out_shape=(jax.ShapeDtypeStruct((G, EMB), jnp.float32),
               jax.ShapeDtypeStruct((G, EMB), jnp.float32),
               jax.ShapeDtypeStruct((1, 1), jnp.float32)),
)

BLKN = 2000
NB = N // BLKN


def _softplus(t):
    return jnp.maximum(t, 0.0) + jnp.log1p(jnp.exp(-jnp.abs(t)))


def _loss_body(x0, x1, x2, x3, x4, gi_ref, genc_ref,
               lw1, lb1, lw2, lb2, lsw, lsb, epos_ref, eneg_ref):
    i = pl.program_id(0)
    Mb = jnp.concatenate([x0[...], x1[...], x2[...], x3[...], x4[...]],
                         axis=1)
    h = jnp.maximum(jnp.dot(Mb, lw1[...],
                            preferred_element_type=jnp.float32) + lb1[...],
                    0.0)
    h = jnp.maximum(jnp.dot(h, lw2[...],
                            preferred_element_type=jnp.float32) + lb2[...],
                    0.0)
    lenc = h + jnp.dot(Mb, lsw[...],
                       preferred_element_type=jnp.float32) + lsb[...]
    res = lax.dot_general(lenc, genc_ref[...], (((1,), (1,)), ((), ())),
                          preferred_element_type=jnp.float32)
    seg = lax.broadcasted_iota(jnp.int32, (BLKN, G), 1)
    pos = (gi_ref[...] == seg).astype(jnp.float32)
    rp = res * pos
    ep = jnp.sum(LOG2 - _softplus(-rp))
    qn = res * (1.0 - pos)
    en = jnp.sum(_softplus(-qn) + qn - LOG2)

    @pl.when(i == 0)
    def _init():
        epos_ref[...] = jnp.zeros((1, 1), jnp.float32)
        eneg_ref[...] = jnp.zeros((1, 1), jnp.float32)

    epos_ref[...] = epos_ref[...] + ep
    eneg_ref[...] = eneg_ref[...] + en


_loss = pl.pallas_call(
    _loss_body,
    grid=(NB,),
    in_specs=[pl.BlockSpec((BLKN, D), lambda i: (i, 0))] * 5
    + [pl.BlockSpec((BLKN, 1), lambda i: (i, 0)),
       pl.BlockSpec((G, EMB), lambda i: (0, 0)),
       pl.BlockSpec((EMB, EMB), lambda i: (0, 0)),
       pl.BlockSpec((1, EMB), lambda i: (0, 0)),
       pl.BlockSpec((EMB, EMB), lambda i: (0, 0)),
       pl.BlockSpec((1, EMB), lambda i: (0, 0)),
       pl.BlockSpec((EMB, EMB), lambda i: (0, 0)),
       pl.BlockSpec((1, EMB), lambda i: (0, 0))],
    out_specs=(pl.BlockSpec((1, 1), lambda i: (0, 0)),
               pl.BlockSpec((1, 1), lambda i: (0, 0))),
    out_shape=(jax.ShapeDtypeStruct((1, 1), jnp.float32),
               jax.ShapeDtypeStruct((1, 1), jnp.float32)),
)


def kernel(node_features, edge_index, graph_index, prior_noise, params):
    p = params
    src = edge_index[0].astype(jnp.int32)
    dst = edge_index[1].astype(jnp.int32)
    pad = EP - E
    pad_i = jnp.arange(pad, dtype=jnp.int32)
    src2 = jnp.concatenate([src, pad_i % N]).reshape(NW, NBLK_W, BLK)
    dst2 = jnp.concatenate([dst, N + (pad_i % 16)]).reshape(NW, NBLK_W, BLK)
    gi = graph_index.astype(jnp.int32)
    gi_row = gi.reshape(1, N)
    gi_col = gi.reshape(N, 1)

    z = _pre(node_features, p['conv0_w1'])
    xs = []
    for l in range(NLAYERS):
        apart = _agg(z, src2, dst2)[:, :N, :]
        b1 = p['conv%d_b1' % l].reshape(1, D)
        w2 = p['conv%d_w2' % l]
        b2 = p['conv%d_b2' % l].reshape(1, D)
        gm = p['bn%d_gamma' % l].reshape(1, D)
        bt = p['bn%d_beta' % l].reshape(1, D)
        if l < NLAYERS - 1:
            x, z = _layer(z, apart[0], apart[1], b1, w2, b2, gm, bt,
                          p['conv%d_w1' % (l + 1)])
        else:
            x = _layer_last(z, apart[0], apart[1], b1, w2, b2, gm, bt)
        xs.append(x)

    y, genc, prior = _pool(
        xs[0], xs[1], xs[2], xs[3], xs[4], gi_row, prior_noise,
        p['gd_w1'], p['gd_b1'].reshape(1, EMB),
        p['gd_w2'], p['gd_b2'].reshape(1, EMB),
        p['gd_skip_w'], p['gd_skip_b'].reshape(1, EMB),
        p['pd_w1'], p['pd_b1'].reshape(1, EMB),
        p['pd_w2'], p['pd_b2'].reshape(1, 1))
    epos, eneg = _loss(
        xs[0], xs[1], xs[2], xs[3], xs[4], gi_col, genc,
        p['ld_w1'], p['ld_b1'].reshape(1, EMB),
        p['ld_w2'], p['ld_b2'].reshape(1, EMB),
        p['ld_skip_w'], p['ld_skip_b'].reshape(1, EMB))
    e_pos = epos[0, 0] / N
    e_neg = eneg[0, 0] / (N * (G - 1))
    return (e_neg - e_pos) + prior[0, 0]


# speedup vs baseline: 12.6729x; 1.0112x over previous
import functools

import jax
import jax.numpy as jnp
import numpy as np
from jax import lax
from jax.experimental import pallas as pl
from jax.experimental.pallas import tpu as pltpu
from jax.experimental.pallas import tpu_sc as plsc

N = 10000
E = 320000
F = 128
D = 64
NLAYERS = 5
EMB = D * NLAYERS
G = 200
GAMMA = 0.1
LOG2 = float(np.log(2.0))

NC = 2
NS = 16
NW = NC * NS
BLK = 128
NBLK_W = 80
EP = NW * NBLK_W * BLK
NPAD = 10240
ZROWS = NPAD // NS
SROWS = N // NS
NBUF = 8
NGRP = NBLK_W // NBUF


def _agg_body(z_hbm, src_hbm, dst_hbm, out_hbm,
              src_v, dst_v, rows_v, acc_sh, gsem, ssem):
    c = lax.axis_index("c")
    s = lax.axis_index("s")
    wid = c * NS + s
    zv = jnp.zeros((16,), jnp.float32)

    def zrow(r, carry):
        for k in range(D // 16):
            rows_v[0, r, pl.ds(k * 16, 16)] = zv
        return carry

    lax.fori_loop(0, BLK, zrow, 0)
    for t in range(ZROWS // BLK):
        pltpu.sync_copy(rows_v.at[0],
                        acc_sh.at[pl.ds(s * ZROWS + t * BLK, BLK)])
    pltpu.sync_copy(src_hbm.at[wid], src_v)
    pltpu.sync_copy(dst_hbm.at[wid], dst_v)
    plsc.subcore_barrier()

    for b in range(NBUF):
        pltpu.async_copy(z_hbm.at[src_v.at[b]], rows_v.at[b], gsem)

    def step(j, carry):
        b = lax.rem(j, NBUF)
        pltpu.make_async_copy(z_hbm.at[src_v.at[j]], rows_v.at[b],
                              gsem).wait()
        pltpu.async_copy(rows_v.at[b], acc_sh.at[dst_v.at[j]], ssem,
                         add=True)
        k = j - 1

        @pl.when(jnp.logical_and(k >= 0, k + NBUF < NBLK_W))
        def _refill():
            kb = lax.rem(k, NBUF)
            pltpu.make_async_copy(rows_v.at[kb], acc_sh.at[dst_v.at[k]],
                                  ssem).wait()
            pltpu.async_copy(z_hbm.at[src_v.at[k + NBUF]], rows_v.at[kb],
                             gsem)

        return carry

    lax.fori_loop(0, NBLK_W, step, 0)

    def drain(j, carry):
        b = lax.rem(j, NBUF)
        pltpu.make_async_copy(rows_v.at[b], acc_sh.at[dst_v.at[j]],
                              ssem).wait()
        return carry

    lax.fori_loop(NBLK_W - NBUF, NBLK_W, drain, 0)
    plsc.subcore_barrier()
    pltpu.sync_copy(acc_sh.at[pl.ds(s * ZROWS, ZROWS)],
                    out_hbm.at[c].at[pl.ds(s * ZROWS, ZROWS)])


_agg = pl.kernel(
    _agg_body,
    out_type=jax.ShapeDtypeStruct((NC, NPAD, D), jnp.float32),
    mesh=plsc.VectorSubcoreMesh(core_axis_name="c", subcore_axis_name="s"),
    scratch_types=[
        pltpu.VMEM((NBLK_W, BLK), jnp.int32),
        pltpu.VMEM((NBLK_W, BLK), jnp.int32),
        pltpu.VMEM((NBUF, BLK, D), jnp.float32),
        pltpu.VMEM_SHARED((NPAD, D), jnp.float32),
        pltpu.SemaphoreType.DMA,
        pltpu.SemaphoreType.DMA,
    ],
    compiler_params=pltpu.CompilerParams(use_tc_tiling_on_sc=False),
)


def _pre_body(x_ref, w_ref, o_ref):
    o_ref[...] = jnp.dot(x_ref[...], w_ref[...],
                         preferred_element_type=jnp.float32)


_pre = pl.pallas_call(
    _pre_body,
    out_shape=jax.ShapeDtypeStruct((N, D), jnp.float32),
)


def _bn_mlp(z, a0, a1, b1, w2, b2, gm, bt):
    u = jnp.maximum(z + a0 + a1 + b1, 0.0)
    v = jnp.dot(u, w2, preferred_element_type=jnp.float32) + b2
    xr = jnp.maximum(v, 0.0)
    mean = jnp.mean(xr, axis=0, keepdims=True)
    xc = xr - mean
    var = jnp.mean(xc * xc, axis=0, keepdims=True)
    return xc / jnp.sqrt(var + 1e-5) * gm + bt


def _layer_body(z_ref, a0_ref, a1_ref, b1_ref, w2_ref, b2_ref, gm_ref,
                bt_ref, w1n_ref, x_ref, zn_ref):
    xbn = _bn_mlp(z_ref[...], a0_ref[...], a1_ref[...], b1_ref[...],
                  w2_ref[...], b2_ref[...], gm_ref[...], bt_ref[...])
    x_ref[...] = xbn
    zn_ref[...] = jnp.dot(xbn, w1n_ref[...],
                          preferred_element_type=jnp.float32)


_layer = pl.pallas_call(
    _layer_body,
    out_shape=(jax.ShapeDtypeStruct((N, D), jnp.float32),
               jax.ShapeDtypeStruct((N, D), jnp.float32)),
)


def _layer_last_body(z_ref, a0_ref, a1_ref, b1_ref, w2_ref, b2_ref, gm_ref,
                     bt_ref, x_ref):
    x_ref[...] = _bn_mlp(z_ref[...], a0_ref[...], a1_ref[...], b1_ref[...],
                         w2_ref[...], b2_ref[...], gm_ref[...], bt_ref[...])


_layer_last = pl.pallas_call(
    _layer_last_body,
    out_shape=jax.ShapeDtypeStruct((N, D), jnp.float32),
)


def _sigmoid(t):
    return 1.0 / (1.0 + jnp.exp(-t))


def _pool_body(x0, x1, x2, x3, x4, gi_ref, pn_ref,
               gw1, gb1, gw2, gb2, gsw, gsb, pw1, pb1, pw2, pb2,
               y_ref, genc_ref, prior_ref):
    M = jnp.concatenate([x0[...], x1[...], x2[...], x3[...], x4[...]],
                        axis=1)
    seg = lax.broadcasted_iota(jnp.int32, (G, N), 0)
    pt = (gi_ref[...] == seg).astype(jnp.float32)
    y = lax.dot_general(pt, M, (((1,), (0,)), ((), ())),
                        precision=lax.Precision.HIGHEST,
                        preferred_element_type=jnp.float32)
    y_ref[...] = y
    h = jnp.maximum(jnp.dot(y, gw1[...],
                            preferred_element_type=jnp.float32) + gb1[...],
                    0.0)
    h = jnp.maximum(jnp.dot(h, gw2[...],
                            preferred_element_type=jnp.float32) + gb2[...],
                    0.0)
    genc_ref[...] = h + jnp.dot(y, gsw[...],
                                preferred_element_type=jnp.float32) + gsb[...]

    def prior_d(t):
        hh = _sigmoid(jnp.dot(t, pw1[...],
                              preferred_element_type=jnp.float32) + pb1[...])
        return _sigmoid(jnp.dot(hh, pw2[...],
                                preferred_element_type=jnp.float32) + pb2[...])

    term_a = jnp.mean(jnp.log(prior_d(pn_ref[...])))
    term_b = jnp.mean(jnp.log(1.0 - prior_d(y)))
    prior_ref[...] = jnp.full((1, 1), -(term_a + term_b) * GAMMA,
                              jnp.float32)


_pool = pl.pallas_call(
    _pool_body,
    out_shape=(jax.ShapeDtypeStruct((G, EMB), jnp.float32),
               jax.ShapeDtypeStruct((G, EMB), jnp.float32),
               jax.ShapeDtypeStruct((1, 1), jnp.float32)),
)

BLKN = 2000
NB = N // BLKN


def _softplus(t):
    return jnp.maximum(t, 0.0) + jnp.log1p(jnp.exp(-jnp.abs(t)))


def _loss_body(x0, x1, x2, x3, x4, gi_ref, genc_ref,
               lw1, lb1, lw2, lb2, lsw, lsb, epos_ref, eneg_ref):
    i = pl.program_id(0)
    Mb = jnp.concatenate([x0[...], x1[...], x2[...], x3[...], x4[...]],
                         axis=1)
    h = jnp.maximum(jnp.dot(Mb, lw1[...],
                            preferred_element_type=jnp.float32) + lb1[...],
                    0.0)
    h = jnp.maximum(jnp.dot(h, lw2[...],
                            preferred_element_type=jnp.float32) + lb2[...],
                    0.0)
    lenc = h + jnp.dot(Mb, lsw[...],
                       preferred_element_type=jnp.float32) + lsb[...]
    res = lax.dot_general(genc_ref[...], lenc, (((1,), (1,)), ((), ())),
                          preferred_element_type=jnp.float32)
    seg = lax.broadcasted_iota(jnp.int32, (G, BLKN), 0)
    pos = (gi_ref[0] == seg).astype(jnp.float32)
    rp = res * pos
    ep = jnp.sum(LOG2 - _softplus(-rp))
    qn = res * (1.0 - pos)
    en = jnp.sum(_softplus(-qn) + qn - LOG2)

    @pl.when(i == 0)
    def _init():
        epos_ref[...] = jnp.zeros((1, 1), jnp.float32)
        eneg_ref[...] = jnp.zeros((1, 1), jnp.float32)

    epos_ref[...] = epos_ref[...] + ep
    eneg_ref[...] = eneg_ref[...] + en


_loss = pl.pallas_call(
    _loss_body,
    grid=(NB,),
    in_specs=[pl.BlockSpec((BLKN, D), lambda i: (i, 0))] * 5
    + [pl.BlockSpec((1, 1, BLKN), lambda i: (i, 0, 0)),
       pl.BlockSpec((G, EMB), lambda i: (0, 0)),
       pl.BlockSpec((EMB, EMB), lambda i: (0, 0)),
       pl.BlockSpec((1, EMB), lambda i: (0, 0)),
       pl.BlockSpec((EMB, EMB), lambda i: (0, 0)),
       pl.BlockSpec((1, EMB), lambda i: (0, 0)),
       pl.BlockSpec((EMB, EMB), lambda i: (0, 0)),
       pl.BlockSpec((1, EMB), lambda i: (0, 0))],
    out_specs=(pl.BlockSpec((1, 1), lambda i: (0, 0)),
               pl.BlockSpec((1, 1), lambda i: (0, 0))),
    out_shape=(jax.ShapeDtypeStruct((1, 1), jnp.float32),
               jax.ShapeDtypeStruct((1, 1), jnp.float32)),
)


def kernel(node_features, edge_index, graph_index, prior_noise, params):
    p = params
    src = edge_index[0].astype(jnp.int32)
    dst = edge_index[1].astype(jnp.int32)
    pad = EP - E
    pad_i = jnp.arange(pad, dtype=jnp.int32)
    src2 = jnp.concatenate([src, pad_i % N]).reshape(NW, NBLK_W, BLK)
    dst2 = jnp.concatenate([dst, N + (pad_i % 16)]).reshape(NW, NBLK_W, BLK)
    gi_row = graph_index.astype(jnp.int32).reshape(1, N)

    z = _pre(node_features, p['conv0_w1'])
    xs = []
    for l in range(NLAYERS):
        apart = _agg(z, src2, dst2)[:, :N, :]
        b1 = p['conv%d_b1' % l].reshape(1, D)
        w2 = p['conv%d_w2' % l]
        b2 = p['conv%d_b2' % l].reshape(1, D)
        gm = p['bn%d_gamma' % l].reshape(1, D)
        bt = p['bn%d_beta' % l].reshape(1, D)
        if l < NLAYERS - 1:
            x, z = _layer(z, apart[0], apart[1], b1, w2, b2, gm, bt,
                          p['conv%d_w1' % (l + 1)])
        else:
            x = _layer_last(z, apart[0], apart[1], b1, w2, b2, gm, bt)
        xs.append(x)

    y, genc, prior = _pool(
        xs[0], xs[1], xs[2], xs[3], xs[4], gi_row, prior_noise,
        p['gd_w1'], p['gd_b1'].reshape(1, EMB),
        p['gd_w2'], p['gd_b2'].reshape(1, EMB),
        p['gd_skip_w'], p['gd_skip_b'].reshape(1, EMB),
        p['pd_w1'], p['pd_b1'].reshape(1, EMB),
        p['pd_w2'], p['pd_b2'].reshape(1, 1))
    epos, eneg = _loss(
        xs[0], xs[1], xs[2], xs[3], xs[4], gi_row.reshape(NB, 1, BLKN), genc,
        p['ld_w1'], p['ld_b1'].reshape(1, EMB),
        p['ld_w2'], p['ld_b2'].reshape(1, EMB),
        p['ld_skip_w'], p['ld_skip_b'].reshape(1, EMB))
    e_pos = epos[0, 0] / N
    e_neg = eneg[0, 0] / (N * (G - 1))
    return (e_neg - e_pos) + prior[0, 0]


# speedup vs baseline: 12.6783x; 1.0004x over previous
import functools

import jax
import jax.numpy as jnp
import numpy as np
from jax import lax
from jax.experimental import pallas as pl
from jax.experimental.pallas import tpu as pltpu
from jax.experimental.pallas import tpu_sc as plsc

N = 10000
E = 320000
F = 128
D = 64
NLAYERS = 5
EMB = D * NLAYERS
G = 200
GAMMA = 0.1
LOG2 = float(np.log(2.0))

NC = 2
NS = 16
NW = NC * NS
BLK = 128
NBLK_W = 80
EP = NW * NBLK_W * BLK
NPAD = 10240
ZROWS = NPAD // NS

NBUF = 8


def _agg_body(z_hbm, src_hbm, dst_hbm, out_hbm,
              src_v, dst_v, rows_v, acc_sh, gsem, ssem):
    c = lax.axis_index("c")
    s = lax.axis_index("s")
    wid = c * NS + s
    zv = jnp.zeros((16,), jnp.float32)

    def zrow(r, carry):
        for k in range(D // 16):
            rows_v[0, r, pl.ds(k * 16, 16)] = zv
        return carry

    lax.fori_loop(0, BLK, zrow, 0)
    for t in range(ZROWS // BLK):
        pltpu.sync_copy(rows_v.at[0],
                        acc_sh.at[pl.ds(s * ZROWS + t * BLK, BLK)])
    pltpu.sync_copy(src_hbm.at[wid], src_v)
    pltpu.sync_copy(dst_hbm.at[wid], dst_v)
    plsc.subcore_barrier()

    for b in range(NBUF):
        pltpu.async_copy(z_hbm.at[src_v.at[b]], rows_v.at[b], gsem)

    def step(j, carry):
        b = lax.rem(j, NBUF)
        pltpu.make_async_copy(z_hbm.at[src_v.at[j]], rows_v.at[b],
                              gsem).wait()
        pltpu.async_copy(rows_v.at[b], acc_sh.at[dst_v.at[j]], ssem,
                         add=True)
        k = j - 1

        @pl.when(jnp.logical_and(k >= 0, k + NBUF < NBLK_W))
        def _refill():
            kb = lax.rem(k, NBUF)
            pltpu.make_async_copy(rows_v.at[kb], acc_sh.at[dst_v.at[k]],
                                  ssem).wait()
            pltpu.async_copy(z_hbm.at[src_v.at[k + NBUF]], rows_v.at[kb],
                             gsem)

        return carry

    lax.fori_loop(0, NBLK_W, step, 0)

    def drain(j, carry):
        b = lax.rem(j, NBUF)
        pltpu.make_async_copy(rows_v.at[b], acc_sh.at[dst_v.at[j]],
                              ssem).wait()
        return carry

    lax.fori_loop(NBLK_W - NBUF, NBLK_W, drain, 0)
    plsc.subcore_barrier()
    pltpu.sync_copy(acc_sh.at[pl.ds(s * ZROWS, ZROWS)],
                    out_hbm.at[c].at[pl.ds(s * ZROWS, ZROWS)])


_agg = pl.kernel(
    _agg_body,
    out_type=jax.ShapeDtypeStruct((NC, NPAD, D), jnp.float32),
    mesh=plsc.VectorSubcoreMesh(core_axis_name="c", subcore_axis_name="s"),
    scratch_types=[
        pltpu.VMEM((NBLK_W, BLK), jnp.int32),
        pltpu.VMEM((NBLK_W, BLK), jnp.int32),
        pltpu.VMEM((NBUF, BLK, D), jnp.float32),
        pltpu.VMEM_SHARED((NPAD, D), jnp.float32),
        pltpu.SemaphoreType.DMA,
        pltpu.SemaphoreType.DMA,
    ],
    compiler_params=pltpu.CompilerParams(use_tc_tiling_on_sc=False),
)


def _pre_body(x_ref, w_ref, o_ref):
    o_ref[...] = jnp.dot(x_ref[...], w_ref[...],
                         preferred_element_type=jnp.float32)


_pre = pl.pallas_call(
    _pre_body,
    out_shape=jax.ShapeDtypeStruct((N, D), jnp.float32),
)


def _bn_mlp(z, a0, a1, b1, w2, b2, gm, bt):
    u = jnp.maximum(z + a0 + a1 + b1, 0.0)
    v = jnp.dot(u, w2, preferred_element_type=jnp.float32) + b2
    xr = jnp.maximum(v, 0.0)
    mean = jnp.mean(xr, axis=0, keepdims=True)
    xc = xr - mean
    var = jnp.mean(xc * xc, axis=0, keepdims=True)
    return xc / jnp.sqrt(var + 1e-5) * gm + bt


def _layer_body(z_ref, a0_ref, a1_ref, b1_ref, w2_ref, b2_ref, gm_ref,
                bt_ref, w1n_ref, x_ref, zn_ref):
    xbn = _bn_mlp(z_ref[...], a0_ref[...], a1_ref[...], b1_ref[...],
                  w2_ref[...], b2_ref[...], gm_ref[...], bt_ref[...])
    x_ref[...] = xbn
    zn_ref[...] = jnp.dot(xbn, w1n_ref[...],
                          preferred_element_type=jnp.float32)


_layer = pl.pallas_call(
    _layer_body,
    out_shape=(jax.ShapeDtypeStruct((N, D), jnp.float32),
               jax.ShapeDtypeStruct((N, D), jnp.float32)),
)


def _layer_last_body(z_ref, a0_ref, a1_ref, b1_ref, w2_ref, b2_ref, gm_ref,
                     bt_ref, x_ref):
    x_ref[...] = _bn_mlp(z_ref[...], a0_ref[...], a1_ref[...], b1_ref[...],
                         w2_ref[...], b2_ref[...], gm_ref[...], bt_ref[...])


_layer_last = pl.pallas_call(
    _layer_last_body,
    out_shape=jax.ShapeDtypeStruct((N, D), jnp.float32),
)


def _sigmoid(t):
    return 1.0 / (1.0 + jnp.exp(-t))


def _pool_body(x0, x1, x2, x3, x4, gi_ref, pn_ref,
               gw1, gb1, gw2, gb2, gsw, gsb, pw1, pb1, pw2, pb2,
               y_ref, genc_ref, prior_ref):
    M = jnp.concatenate([x0[...], x1[...], x2[...], x3[...], x4[...]],
                        axis=1)
    seg = lax.broadcasted_iota(jnp.int32, (G, N), 0)
    pt = (gi_ref[...] == seg).astype(jnp.float32)
    y = lax.dot_general(pt, M, (((1,), (0,)), ((), ())),
                        precision=lax.Precision.HIGHEST,
                        preferred_element_type=jnp.float32)
    y_ref[...] = y
    h = jnp.maximum(jnp.dot(y, gw1[...],
                            preferred_element_type=jnp.float32) + gb1[...],
                    0.0)
    h = jnp.maximum(jnp.dot(h, gw2[...],
                            preferred_element_type=jnp.float32) + gb2[...],
                    0.0)
    genc_ref[...] = h + jnp.dot(y, gsw[...],
                                preferred_element_type=jnp.float32) + gsb[...]

    def prior_d(t):
        hh = _sigmoid(jnp.dot(t, pw1[...],
                              preferred_element_type=jnp.float32) + pb1[...])
        return _sigmoid(jnp.dot(hh, pw2[...],
                                preferred_element_type=jnp.float32) + pb2[...])

    term_a = jnp.mean(jnp.log(prior_d(pn_ref[...])))
    term_b = jnp.mean(jnp.log(1.0 - prior_d(y)))
    prior_ref[...] = jnp.full((1, 1), -(term_a + term_b) * GAMMA,
                              jnp.float32)


_pool = pl.pallas_call(
    _pool_body,
    out_shape=(jax.ShapeDtypeStruct((G, EMB), jnp.float32),
               jax.ShapeDtypeStruct((G, EMB), jnp.float32),
               jax.ShapeDtypeStruct((1, 1), jnp.float32)),
)

BLKN = 2000
NB = N // BLKN


def _softplus(t):
    return jnp.maximum(t, 0.0) + jnp.log1p(jnp.exp(-jnp.abs(t)))


def _loss_body(x0, x1, x2, x3, x4, gi_ref, genc_ref,
               lw1, lb1, lw2, lb2, lsw, lsb, epos_ref, eneg_ref):
    i = pl.program_id(0)
    Mb = jnp.concatenate([x0[...], x1[...], x2[...], x3[...], x4[...]],
                         axis=1)
    h = jnp.maximum(jnp.dot(Mb, lw1[...],
                            preferred_element_type=jnp.float32) + lb1[...],
                    0.0)
    h = jnp.maximum(jnp.dot(h, lw2[...],
                            preferred_element_type=jnp.float32) + lb2[...],
                    0.0)
    lenc = h + jnp.dot(Mb, lsw[...],
                       preferred_element_type=jnp.float32) + lsb[...]
    res = lax.dot_general(genc_ref[...], lenc, (((1,), (1,)), ((), ())),
                          preferred_element_type=jnp.float32)
    seg = lax.broadcasted_iota(jnp.int32, (G, BLKN), 0)
    pos = (gi_ref[0] == seg).astype(jnp.float32)
    rp = res * pos
    ep = jnp.sum(LOG2 - _softplus(-rp))
    qn = res * (1.0 - pos)
    en = jnp.sum(_softplus(-qn) + qn - LOG2)

    @pl.when(i == 0)
    def _init():
        epos_ref[...] = jnp.zeros((1, 1), jnp.float32)
        eneg_ref[...] = jnp.zeros((1, 1), jnp.float32)

    epos_ref[...] = epos_ref[...] + ep
    eneg_ref[...] = eneg_ref[...] + en


_loss = pl.pallas_call(
    _loss_body,
    grid=(NB,),
    in_specs=[pl.BlockSpec((BLKN, D), lambda i: (i, 0))] * 5
    + [pl.BlockSpec((1, 1, BLKN), lambda i: (i, 0, 0)),
       pl.BlockSpec((G, EMB), lambda i: (0, 0)),
       pl.BlockSpec((EMB, EMB), lambda i: (0, 0)),
       pl.BlockSpec((1, EMB), lambda i: (0, 0)),
       pl.BlockSpec((EMB, EMB), lambda i: (0, 0)),
       pl.BlockSpec((1, EMB), lambda i: (0, 0)),
       pl.BlockSpec((EMB, EMB), lambda i: (0, 0)),
       pl.BlockSpec((1, EMB), lambda i: (0, 0))],
    out_specs=(pl.BlockSpec((1, 1), lambda i: (0, 0)),
               pl.BlockSpec((1, 1), lambda i: (0, 0))),
    out_shape=(jax.ShapeDtypeStruct((1, 1), jnp.float32),
               jax.ShapeDtypeStruct((1, 1), jnp.float32)),
)


def kernel(node_features, edge_index, graph_index, prior_noise, params):
    p = params
    src = edge_index[0].astype(jnp.int32)
    dst = edge_index[1].astype(jnp.int32)
    pad = EP - E
    pad_i = jnp.arange(pad, dtype=jnp.int32)
    src2 = jnp.concatenate([src, pad_i % N]).reshape(NW, NBLK_W, BLK)
    dst2 = jnp.concatenate([dst, N + (pad_i % 16)]).reshape(NW, NBLK_W, BLK)
    gi_row = graph_index.astype(jnp.int32).reshape(1, N)

    z = _pre(node_features, p['conv0_w1'])
    xs = []
    for l in range(NLAYERS):
        apart = _agg(z, src2, dst2)[:, :N, :]
        b1 = p['conv%d_b1' % l].reshape(1, D)
        w2 = p['conv%d_w2' % l]
        b2 = p['conv%d_b2' % l].reshape(1, D)
        gm = p['bn%d_gamma' % l].reshape(1, D)
        bt = p['bn%d_beta' % l].reshape(1, D)
        if l < NLAYERS - 1:
            x, z = _layer(z, apart[0], apart[1], b1, w2, b2, gm, bt,
                          p['conv%d_w1' % (l + 1)])
        else:
            x = _layer_last(z, apart[0], apart[1], b1, w2, b2, gm, bt)
        xs.append(x)

    y, genc, prior = _pool(
        xs[0], xs[1], xs[2], xs[3], xs[4], gi_row, prior_noise,
        p['gd_w1'], p['gd_b1'].reshape(1, EMB),
        p['gd_w2'], p['gd_b2'].reshape(1, EMB),
        p['gd_skip_w'], p['gd_skip_b'].reshape(1, EMB),
        p['pd_w1'], p['pd_b1'].reshape(1, EMB),
        p['pd_w2'], p['pd_b2'].reshape(1, 1))
    epos, eneg = _loss(
        xs[0], xs[1], xs[2], xs[3], xs[4], gi_row.reshape(NB, 1, BLKN), genc,
        p['ld_w1'], p['ld_b1'].reshape(1, EMB),
        p['ld_w2'], p['ld_b2'].reshape(1, EMB),
        p['ld_skip_w'], p['ld_skip_b'].reshape(1, EMB))
    e_pos = epos[0, 0] / N
    e_neg = eneg[0, 0] / (N * (G - 1))
    return (e_neg - e_pos) + prior[0, 0]


# speedup vs baseline: 12.9761x; 1.0235x over previous
import functools

import jax
import jax.numpy as jnp
import numpy as np
from jax import lax
from jax.experimental import pallas as pl
from jax.experimental.pallas import tpu as pltpu
from jax.experimental.pallas import tpu_sc as plsc

N = 10000
E = 320000
F = 128
D = 64
NLAYERS = 5
EMB = D * NLAYERS
G = 200
GAMMA = 0.1
LOG2 = float(np.log(2.0))

NC = 2
NS = 16
NW = NC * NS
BLK = 128
NBLK_W = 80
EP = NW * NBLK_W * BLK
NPAD = 10240
ZROWS = NPAD // NS

NBUF = 8


def _agg_body(z_hbm, src_hbm, dst_hbm, out_hbm,
              src_v, dst_v, rows_v, acc_sh, gsem, ssem):
    c = lax.axis_index("c")
    s = lax.axis_index("s")
    wid = c * NS + s
    cp_src = pltpu.async_copy(src_hbm.at[wid], src_v, ssem)
    cp_dst = pltpu.async_copy(dst_hbm.at[wid], dst_v, ssem)
    zv = jnp.zeros((16,), jnp.float32)

    def zrow(r, carry):
        for k in range(D // 16):
            rows_v[0, r, pl.ds(k * 16, 16)] = zv
        return carry

    lax.fori_loop(0, BLK, zrow, 0)
    for t in range(ZROWS // BLK):
        pltpu.sync_copy(rows_v.at[0],
                        acc_sh.at[pl.ds(s * ZROWS + t * BLK, BLK)])
    cp_src.wait()
    cp_dst.wait()
    plsc.subcore_barrier()

    for b in range(NBUF):
        pltpu.async_copy(z_hbm.at[src_v.at[b]], rows_v.at[b], gsem)

    def step(j, carry):
        b = lax.rem(j, NBUF)
        pltpu.make_async_copy(z_hbm.at[src_v.at[j]], rows_v.at[b],
                              gsem).wait()
        pltpu.async_copy(rows_v.at[b], acc_sh.at[dst_v.at[j]], ssem,
                         add=True)
        k = j - 1

        @pl.when(jnp.logical_and(k >= 0, k + NBUF < NBLK_W))
        def _refill():
            kb = lax.rem(k, NBUF)
            pltpu.make_async_copy(rows_v.at[kb], acc_sh.at[dst_v.at[k]],
                                  ssem).wait()
            pltpu.async_copy(z_hbm.at[src_v.at[k + NBUF]], rows_v.at[kb],
                             gsem)

        return carry

    lax.fori_loop(0, NBLK_W, step, 0)

    def drain(j, carry):
        b = lax.rem(j, NBUF)
        pltpu.make_async_copy(rows_v.at[b], acc_sh.at[dst_v.at[j]],
                              ssem).wait()
        return carry

    lax.fori_loop(NBLK_W - NBUF, NBLK_W, drain, 0)
    plsc.subcore_barrier()
    pltpu.sync_copy(acc_sh.at[pl.ds(s * ZROWS, ZROWS)],
                    out_hbm.at[c].at[pl.ds(s * ZROWS, ZROWS)])


_agg = pl.kernel(
    _agg_body,
    out_type=jax.ShapeDtypeStruct((NC, NPAD, D), jnp.float32),
    mesh=plsc.VectorSubcoreMesh(core_axis_name="c", subcore_axis_name="s"),
    scratch_types=[
        pltpu.VMEM((NBLK_W, BLK), jnp.int32),
        pltpu.VMEM((NBLK_W, BLK), jnp.int32),
        pltpu.VMEM((NBUF, BLK, D), jnp.float32),
        pltpu.VMEM_SHARED((NPAD, D), jnp.float32),
        pltpu.SemaphoreType.DMA,
        pltpu.SemaphoreType.DMA,
    ],
    compiler_params=pltpu.CompilerParams(use_tc_tiling_on_sc=False),
)


def _pre_body(x_ref, w_ref, o_ref):
    o_ref[...] = jnp.dot(x_ref[...], w_ref[...],
                         preferred_element_type=jnp.float32)


_pre = pl.pallas_call(
    _pre_body,
    out_shape=jax.ShapeDtypeStruct((N, D), jnp.float32),
)


def _bn_mlp(z, a0, a1, b1, w2, b2, gm, bt):
    u = jnp.maximum(z + a0 + a1 + b1, 0.0)
    v = jnp.dot(u, w2, preferred_element_type=jnp.float32) + b2
    xr = jnp.maximum(v, 0.0)
    mean = jnp.mean(xr, axis=0, keepdims=True)
    xc = xr - mean
    var = jnp.mean(xc * xc, axis=0, keepdims=True)
    return xc / jnp.sqrt(var + 1e-5) * gm + bt


def _layer_body(z_ref, a0_ref, a1_ref, b1_ref, w2_ref, b2_ref, gm_ref,
                bt_ref, w1n_ref, x_ref, zn_ref):
    xbn = _bn_mlp(z_ref[...], a0_ref[...], a1_ref[...], b1_ref[...],
                  w2_ref[...], b2_ref[...], gm_ref[...], bt_ref[...])
    x_ref[...] = xbn
    zn_ref[...] = jnp.dot(xbn, w1n_ref[...],
                          preferred_element_type=jnp.float32)


_layer = pl.pallas_call(
    _layer_body,
    out_shape=(jax.ShapeDtypeStruct((N, D), jnp.float32),
               jax.ShapeDtypeStruct((N, D), jnp.float32)),
)


def _layer_last_body(z_ref, a0_ref, a1_ref, b1_ref, w2_ref, b2_ref, gm_ref,
                     bt_ref, x_ref):
    x_ref[...] = _bn_mlp(z_ref[...], a0_ref[...], a1_ref[...], b1_ref[...],
                         w2_ref[...], b2_ref[...], gm_ref[...], bt_ref[...])


_layer_last = pl.pallas_call(
    _layer_last_body,
    out_shape=jax.ShapeDtypeStruct((N, D), jnp.float32),
)


def _sigmoid(t):
    return 1.0 / (1.0 + jnp.exp(-t))


def _pool_body(x0, x1, x2, x3, x4, gi_ref, pn_ref,
               gw1, gb1, gw2, gb2, gsw, gsb, pw1, pb1, pw2, pb2,
               y_ref, genc_ref, prior_ref):
    M = jnp.concatenate([x0[...], x1[...], x2[...], x3[...], x4[...]],
                        axis=1)
    seg = lax.broadcasted_iota(jnp.int32, (G, N), 0)
    pt = (gi_ref[...] == seg).astype(jnp.float32)
    y = lax.dot_general(pt, M, (((1,), (0,)), ((), ())),
                        precision=lax.Precision.HIGHEST,
                        preferred_element_type=jnp.float32)
    y_ref[...] = y
    h = jnp.maximum(jnp.dot(y, gw1[...],
                            preferred_element_type=jnp.float32) + gb1[...],
                    0.0)
    h = jnp.maximum(jnp.dot(h, gw2[...],
                            preferred_element_type=jnp.float32) + gb2[...],
                    0.0)
    genc_ref[...] = h + jnp.dot(y, gsw[...],
                                preferred_element_type=jnp.float32) + gsb[...]

    def prior_d(t):
        hh = _sigmoid(jnp.dot(t, pw1[...],
                              preferred_element_type=jnp.float32) + pb1[...])
        return _sigmoid(jnp.dot(hh, pw2[...],
                                preferred_element_type=jnp.float32) + pb2[...])

    term_a = jnp.mean(jnp.log(prior_d(pn_ref[...])))
    term_b = jnp.mean(jnp.log(1.0 - prior_d(y)))
    prior_ref[...] = jnp.full((1, 1), -(term_a + term_b) * GAMMA,
                              jnp.float32)


_pool = pl.pallas_call(
    _pool_body,
    out_shape=(jax.ShapeDtypeStruct((G, EMB), jnp.float32),
               jax.ShapeDtypeStruct((G, EMB), jnp.float32),
               jax.ShapeDtypeStruct((1, 1), jnp.float32)),
)

BLKN = 2000
NB = N // BLKN


def _softplus(t):
    return jnp.maximum(t, 0.0) + jnp.log1p(jnp.exp(-jnp.abs(t)))


def _loss_body(x0, x1, x2, x3, x4, gi_ref, genc_ref,
               lw1, lb1, lw2, lb2, lsw, lsb, epos_ref, eneg_ref):
    i = pl.program_id(0)
    Mb = jnp.concatenate([x0[...], x1[...], x2[...], x3[...], x4[...]],
                         axis=1)
    h = jnp.maximum(jnp.dot(Mb, lw1[...],
                            preferred_element_type=jnp.float32) + lb1[...],
                    0.0)
    h = jnp.maximum(jnp.dot(h, lw2[...],
                            preferred_element_type=jnp.float32) + lb2[...],
                    0.0)
    lenc = h + jnp.dot(Mb, lsw[...],
                       preferred_element_type=jnp.float32) + lsb[...]
    res = lax.dot_general(genc_ref[...], lenc, (((1,), (1,)), ((), ())),
                          preferred_element_type=jnp.float32)
    seg = lax.broadcasted_iota(jnp.int32, (G, BLKN), 0)
    pos = (gi_ref[0] == seg).astype(jnp.float32)
    rp = res * pos
    ep = jnp.sum(LOG2 - _softplus(-rp))
    qn = res * (1.0 - pos)
    en = jnp.sum(_softplus(-qn) + qn - LOG2)

    @pl.when(i == 0)
    def _init():
        epos_ref[...] = jnp.zeros((1, 1), jnp.float32)
        eneg_ref[...] = jnp.zeros((1, 1), jnp.float32)

    epos_ref[...] = epos_ref[...] + ep
    eneg_ref[...] = eneg_ref[...] + en


_loss = pl.pallas_call(
    _loss_body,
    grid=(NB,),
    in_specs=[pl.BlockSpec((BLKN, D), lambda i: (i, 0))] * 5
    + [pl.BlockSpec((1, 1, BLKN), lambda i: (i, 0, 0)),
       pl.BlockSpec((G, EMB), lambda i: (0, 0)),
       pl.BlockSpec((EMB, EMB), lambda i: (0, 0)),
       pl.BlockSpec((1, EMB), lambda i: (0, 0)),
       pl.BlockSpec((EMB, EMB), lambda i: (0, 0)),
       pl.BlockSpec((1, EMB), lambda i: (0, 0)),
       pl.BlockSpec((EMB, EMB), lambda i: (0, 0)),
       pl.BlockSpec((1, EMB), lambda i: (0, 0))],
    out_specs=(pl.BlockSpec((1, 1), lambda i: (0, 0)),
               pl.BlockSpec((1, 1), lambda i: (0, 0))),
    out_shape=(jax.ShapeDtypeStruct((1, 1), jnp.float32),
               jax.ShapeDtypeStruct((1, 1), jnp.float32)),
)


def kernel(node_features, edge_index, graph_index, prior_noise, params):
    p = params
    src = edge_index[0].astype(jnp.int32)
    dst = edge_index[1].astype(jnp.int32)
    pad = EP - E
    pad_i = jnp.arange(pad, dtype=jnp.int32)
    src2 = jnp.concatenate([src, pad_i % N]).reshape(NW, NBLK_W, BLK)
    dst2 = jnp.concatenate([dst, N + (pad_i % 16)]).reshape(NW, NBLK_W, BLK)
    gi_row = graph_index.astype(jnp.int32).reshape(1, N)

    z = _pre(node_features, p['conv0_w1'])
    xs = []
    for l in range(NLAYERS):
        apart = _agg(z, src2, dst2)[:, :N, :]
        b1 = p['conv%d_b1' % l].reshape(1, D)
        w2 = p['conv%d_w2' % l]
        b2 = p['conv%d_b2' % l].reshape(1, D)
        gm = p['bn%d_gamma' % l].reshape(1, D)
        bt = p['bn%d_beta' % l].reshape(1, D)
        if l < NLAYERS - 1:
            x, z = _layer(z, apart[0], apart[1], b1, w2, b2, gm, bt,
                          p['conv%d_w1' % (l + 1)])
        else:
            x = _layer_last(z, apart[0], apart[1], b1, w2, b2, gm, bt)
        xs.append(x)

    y, genc, prior = _pool(
        xs[0], xs[1], xs[2], xs[3], xs[4], gi_row, prior_noise,
        p['gd_w1'], p['gd_b1'].reshape(1, EMB),
        p['gd_w2'], p['gd_b2'].reshape(1, EMB),
        p['gd_skip_w'], p['gd_skip_b'].reshape(1, EMB),
        p['pd_w1'], p['pd_b1'].reshape(1, EMB),
        p['pd_w2'], p['pd_b2'].reshape(1, 1))
    epos, eneg = _loss(
        xs[0], xs[1], xs[2], xs[3], xs[4], gi_row.reshape(NB, 1, BLKN), genc,
        p['ld_w1'], p['ld_b1'].reshape(1, EMB),
        p['ld_w2'], p['ld_b2'].reshape(1, EMB),
        p['ld_skip_w'], p['ld_skip_b'].reshape(1, EMB))
    e_pos = epos[0, 0] / N
    e_neg = eneg[0, 0] / (N * (G - 1))
    return (e_neg - e_pos) + prior[0, 0]
